# baseline (speedup 1.0000x reference)
import jax
import jax.numpy as jnp
from jax import lax
from jax.experimental import pallas as pl
from jax.experimental.pallas import tpu as pltpu
from jax.experimental.pallas import tpu_sc as plsc

T = 4096
D = 2048
R = 4
L = 4096
B = R * L
NC, NS = 2, 16
NW = NC * NS
RPT = T // NW
C = 16
NSL = RPT // C
CAP = B + 16


def _routed_body(idx_hbm, table_hbm, out_hbm,
                 idx_v, own_l, slice_l, buf0, buf1,
                 isem, gsem0, gsem1, ssem):
    wid = lax.axis_index("s") * NC + lax.axis_index("c")
    tbase = wid * RPT

    bufs = (buf0, buf1)
    gsems = (gsem0, gsem1)

    pltpu.async_copy(table_hbm.at[pl.ds(tbase, C)], buf0, gsem0)
    pltpu.async_copy(table_hbm.at[pl.ds(tbase + C, C)], buf1, gsem1)
    pltpu.async_copy(idx_hbm, idx_v, isem)
    pltpu.make_async_copy(idx_hbm, idx_v, isem).wait()

    lanes = lax.iota(jnp.int32, 16)
    row_consts = [
        lax.shift_left(r * L + lanes, 7) for r in range(R)
    ]

    ones = jnp.full((16,), 1, jnp.int32)
    zeros = jnp.full((16,), 0, jnp.int32)

    def scan_step(i, off_vec):
        ibits = lax.shift_left(i, 11)
        for r in range(R):
            x = idx_v[r, pl.ds(i * 16, 16)]
            rel = x - tbase
            m = (rel >= 0) & (rel < RPT)
            mi = lax.select(m, ones, zeros)
            rank = plsc.cumsum(mi) - mi
            packed = (row_consts[r] | rel) + ibits
            plsc.store_scatter(own_l, [off_vec + rank], packed, mask=m)
            off_vec = off_vec + plsc.all_reduce_population_count(m)
        return off_vec

    n_own = lax.fori_loop(0, L // 16, scan_step, zeros)[0]

    def do_slice(sl, buf, gsem):
        lo = sl * C

        def filt_step(k, off_vec):
            v = own_l[pl.ds(k * 16, 16)]
            r = v & (RPT - 1)
            m = (r >= lo) & (r < lo + C) & (k * 16 + lanes < n_own)
            mi = lax.select(m, ones, zeros)
            rank = plsc.cumsum(mi) - mi
            plsc.store_scatter(slice_l, [off_vec + rank], v, mask=m)
            return off_vec + plsc.all_reduce_population_count(m)

        n_sl = lax.fori_loop(0, (n_own + 15) // 16, filt_step, zeros)[0]

        pltpu.make_async_copy(
            table_hbm.at[pl.ds(tbase + lo, C)], buf, gsem).wait()

        def fire(k, carry):
            v16 = slice_l[pl.ds(k * 16, 16)]
            for j in range(16):
                @pl.when(k * 16 + j < n_sl)
                def _():
                    v = v16[j]
                    pos = lax.shift_right_logical(v, 7)
                    row = (v & (RPT - 1)) - lo
                    pltpu.make_async_copy(
                        buf.at[pl.ds(row, 1)], out_hbm.at[pl.ds(pos, 1)],
                        ssem).start()
            return carry

        lax.fori_loop(0, (n_sl + 15) // 16, fire, 0)

        def drain16(j, carry):
            pltpu.make_async_copy(
                table_hbm.at[pl.ds(0, C)], buf, ssem).wait()
            return carry

        def drain1(j, carry):
            pltpu.make_async_copy(
                table_hbm.at[pl.ds(0, 1)], buf.at[pl.ds(0, 1)], ssem).wait()
            return carry

        lax.fori_loop(0, n_sl // 16, drain16, 0)
        lax.fori_loop(0, n_sl & 15, drain1, 0)

        @pl.when(sl + 2 < NSL)
        def _():
            pltpu.async_copy(
                table_hbm.at[pl.ds(tbase + (sl + 2) * C, C)], buf, gsem)

    def slice_pair(s2, carry):
        for b in range(2):
            do_slice(2 * s2 + b, bufs[b], gsems[b])
        return carry

    lax.fori_loop(0, NSL // 2, slice_pair, 0)


def kernel(pos, pos_embedding):
    mesh = plsc.VectorSubcoreMesh(core_axis_name="c", subcore_axis_name="s")
    out = pl.kernel(
        _routed_body,
        mesh=mesh,
        compiler_params=pltpu.CompilerParams(needs_layout_passes=False),
        out_type=jax.ShapeDtypeStruct((B, D), jnp.float32),
        scratch_types=[
            pltpu.VMEM((R, L), jnp.int32),
            pltpu.VMEM((CAP,), jnp.int32),
            pltpu.VMEM((CAP,), jnp.int32),
            pltpu.VMEM((C, D), jnp.float32),
            pltpu.VMEM((C, D), jnp.float32),
            pltpu.SemaphoreType.DMA,
            pltpu.SemaphoreType.DMA,
            pltpu.SemaphoreType.DMA,
            pltpu.SemaphoreType.DMA,
        ],
    )(pos, pos_embedding)
    return out.reshape(pos.shape[0], pos.shape[1], D)
